# baseline (speedup 1.0000x reference)
import functools

import jax
import jax.numpy as jnp
from jax import lax
from jax.experimental import pallas as pl
from jax.experimental.pallas import tpu as pltpu
from jax.experimental.pallas import tpu_sc as plsc

N_ITEMS = 100000
N_CAT = 1000
FDIM = 512
K = 64
K2 = 32
B = 16384
LAMBDA_W = 0.01
LAMBDA_B = 0.01

NC, NS = 2, 16
NW = NC * NS
W = B // NW
CF = 64
CT = 128
TAB = 128
PBLK = 2000


def _pack_body(gu_r, tu_r, gi_r, bi_r, ic_r, ut_r, it_r):
    f32 = jnp.float32
    blk = gu_r.shape[0]
    z32 = jnp.zeros((blk, TAB - K - K2), f32)
    ut_r[...] = jnp.concatenate([gu_r[...], tu_r[...], z32], axis=1)
    icf = jax.lax.bitcast_convert_type(ic_r[...], f32)
    z62 = jnp.zeros((blk, TAB - K - 2), f32)
    it_r[...] = jnp.concatenate([gi_r[...], bi_r[...], icf, z62], axis=1)


def _pack_tables(Gu, Tu, Gi, Bi2, IC2):
    f32 = jnp.float32
    G = N_ITEMS // PBLK
    row = lambda b: (b, 0)
    return pl.pallas_call(
        _pack_body,
        grid=(G,),
        in_specs=[
            pl.BlockSpec((PBLK, K), row),
            pl.BlockSpec((PBLK, K2), row),
            pl.BlockSpec((PBLK, K), row),
            pl.BlockSpec((PBLK, 1), row),
            pl.BlockSpec((PBLK, 1), row),
        ],
        out_specs=[pl.BlockSpec((PBLK, TAB), row),
                   pl.BlockSpec((PBLK, TAB), row)],
        out_shape=[jax.ShapeDtypeStruct((N_ITEMS, TAB), f32),
                   jax.ShapeDtypeStruct((N_ITEMS, TAB), f32)],
        compiler_params=pltpu.CompilerParams(
            dimension_semantics=("arbitrary",)),
    )(Gu, Tu, Gi, Bi2, IC2)


def _sc_gather_f(i, j, F):
    mesh = plsc.VectorSubcoreMesh(core_axis_name="c", subcore_axis_name="s")
    f32 = jnp.float32
    out_type = (
        jax.ShapeDtypeStruct((B, FDIM), f32),
        jax.ShapeDtypeStruct((B, FDIM), f32),
    )
    scratch = [
        pltpu.VMEM((W,), jnp.int32),
        pltpu.VMEM((W,), jnp.int32),
        pltpu.VMEM((CF, FDIM), f32),
        pltpu.VMEM((CF, FDIM), f32),
        pltpu.SemaphoreType.DMA,
    ]

    @functools.partial(pl.kernel, out_type=out_type, mesh=mesh,
                       scratch_types=scratch)
    def body(i_h, j_h, F_h, fi_o, fj_o, iv, jv, fi_v, fj_v, sem):
        wid = lax.axis_index("s") * NC + lax.axis_index("c")
        base0 = wid * W
        d = [pltpu.async_copy(i_h.at[pl.ds(base0, W)], iv, sem),
             pltpu.async_copy(j_h.at[pl.ds(base0, W)], jv, sem)]
        for t in d:
            t.wait()

        def chunk(c, carry):
            o = c * CF
            s = pl.ds(o, CF)
            d = [pltpu.async_copy(F_h.at[iv.at[s]], fi_v, sem),
                 pltpu.async_copy(F_h.at[jv.at[s]], fj_v, sem)]
            for t in d:
                t.wait()
            so = pl.ds(base0 + o, CF)
            d = [pltpu.async_copy(fi_v, fi_o.at[so], sem),
                 pltpu.async_copy(fj_v, fj_o.at[so], sem)]
            for t in d:
                t.wait()
            return carry

        lax.fori_loop(0, W // CF, chunk, 0)

    return body(i, j, F)


def _sc_gather_tabs(u, i, j, UserTab, ItemTab):
    mesh = plsc.VectorSubcoreMesh(core_axis_name="c", subcore_axis_name="s")
    f32 = jnp.float32
    out_type = (
        jax.ShapeDtypeStruct((B, TAB), f32),
        jax.ShapeDtypeStruct((B, TAB), f32),
        jax.ShapeDtypeStruct((B, TAB), f32),
    )
    scratch = [
        pltpu.VMEM((W,), jnp.int32),
        pltpu.VMEM((W,), jnp.int32),
        pltpu.VMEM((W,), jnp.int32),
        pltpu.VMEM((CT, TAB), f32),
        pltpu.VMEM((CT, TAB), f32),
        pltpu.VMEM((CT, TAB), f32),
        pltpu.SemaphoreType.DMA,
    ]

    @functools.partial(pl.kernel, out_type=out_type, mesh=mesh,
                       scratch_types=scratch)
    def body(u_h, i_h, j_h, UT_h, IT_h,
             ur_o, ir_o, jr_o, uv, iv, jv, ur_v, ir_v, jr_v, sem):
        wid = lax.axis_index("s") * NC + lax.axis_index("c")
        base0 = wid * W
        d = [pltpu.async_copy(u_h.at[pl.ds(base0, W)], uv, sem),
             pltpu.async_copy(i_h.at[pl.ds(base0, W)], iv, sem),
             pltpu.async_copy(j_h.at[pl.ds(base0, W)], jv, sem)]
        for t in d:
            t.wait()

        def chunk(c, carry):
            o = c * CT
            s = pl.ds(o, CT)
            d = [pltpu.async_copy(UT_h.at[uv.at[s]], ur_v, sem),
                 pltpu.async_copy(IT_h.at[iv.at[s]], ir_v, sem),
                 pltpu.async_copy(IT_h.at[jv.at[s]], jr_v, sem)]
            for t in d:
                t.wait()
            so = pl.ds(base0 + o, CT)
            d = [pltpu.async_copy(ur_v, ur_o.at[so], sem),
                 pltpu.async_copy(ir_v, ir_o.at[so], sem),
                 pltpu.async_copy(jr_v, jr_o.at[so], sem)]
            for t in d:
                t.wait()
            return carry

        lax.fori_loop(0, W // CT, chunk, 0)

    return body(u, i, j, UserTab, ItemTab)


def _tc_body(ur_r, ir_r, jr_r, fi_r, fj_r, E_r, Bp_r, Ic_r, loss_r, auc_r):
    pid = pl.program_id(0)
    f32 = jnp.float32
    ub = ur_r[...]
    ib = ir_r[...]
    jb = jr_r[...]
    gu = ub[:, :K]
    tu = ub[:, K:K + K2]
    gi = ib[:, :K]
    gj = jb[:, :K]
    bi = ib[:, K:K + 1]
    bj = jb[:, K:K + 1]
    ci = jax.lax.bitcast_convert_type(ib[:, K + 1:K + 2], jnp.int32)
    cj = jax.lax.bitcast_convert_type(jb[:, K + 1:K + 2], jnp.int32)

    blk = ub.shape[0]
    cats = jax.lax.broadcasted_iota(jnp.int32, (blk, N_CAT), 1)
    zdiff = (cats == ci).astype(f32) - (cats == cj).astype(f32)
    cfd = jnp.dot(zdiff, Ic_r[...], preferred_element_type=f32)

    fd = fi_r[...] - fj_r[...]
    t = jnp.dot(fd, E_r[...], preferred_element_type=f32)
    g = jnp.dot(fd, Bp_r[...], preferred_element_type=f32)

    ones_k = jnp.ones((K, 1), f32)
    ones_k2 = jnp.ones((K2, 1), f32)
    x = (bi - bj
         + jnp.dot(gu * (gi - gj), ones_k, preferred_element_type=f32)
         + jnp.dot(tu * (t - cfd), ones_k2, preferred_element_type=f32)
         + g)
    pll = jnp.minimum(x, 0.0) - jnp.log1p(jnp.exp(-jnp.abs(x)))
    pauc = (x > 0.0).astype(f32)
    preg = (0.5 * LAMBDA_W * (
                jnp.dot(gu * gu + gi * gi + gj * gj, ones_k,
                        preferred_element_type=f32)
                + jnp.dot(tu * tu, ones_k2, preferred_element_type=f32))
            + 0.5 * LAMBDA_B * (bi * bi + bj * bj))

    @pl.when(pid == 0)
    def _():
        loss_r[0, 0] = 0.0
        auc_r[0, 0] = 0.0

    loss_r[0, 0] += jnp.sum(preg - pll)
    auc_r[0, 0] += jnp.sum(pauc)


def _tc_math(ur, ir, jr, fi, fj, E, Bp, Ic):
    BLK = 512
    G = B // BLK
    f32 = jnp.float32
    row = lambda b: (b, 0)
    full = lambda b: (0, 0)
    grid_spec = pl.GridSpec(
        grid=(G,),
        in_specs=[
            pl.BlockSpec((BLK, TAB), row),
            pl.BlockSpec((BLK, TAB), row),
            pl.BlockSpec((BLK, TAB), row),
            pl.BlockSpec((BLK, FDIM), row),
            pl.BlockSpec((BLK, FDIM), row),
            pl.BlockSpec((FDIM, K2), full),
            pl.BlockSpec((FDIM, 1), full),
            pl.BlockSpec((N_CAT, K2), full),
        ],
        out_specs=[
            pl.BlockSpec((1, 1), full, memory_space=pltpu.SMEM),
            pl.BlockSpec((1, 1), full, memory_space=pltpu.SMEM),
        ],
    )
    loss, auc = pl.pallas_call(
        _tc_body,
        grid_spec=grid_spec,
        out_shape=[jax.ShapeDtypeStruct((1, 1), f32),
                   jax.ShapeDtypeStruct((1, 1), f32)],
        compiler_params=pltpu.CompilerParams(
            dimension_semantics=("arbitrary",)),
    )(ur, ir, jr, fi, fj, E, Bp, Ic)
    return loss[0, 0], auc[0, 0]


def kernel(u, i, j, Bi, Gu, Gi, Tu, Ic, E, Bp, F, IC):
    u = u.astype(jnp.int32)
    i = i.astype(jnp.int32)
    j = j.astype(jnp.int32)
    fi, fj = _sc_gather_f(i, j, F)
    UserTab, ItemTab = _pack_tables(Gu, Tu, Gi, Bi[:, None], IC[:, None])
    ur, ir, jr = _sc_gather_tabs(u, i, j, UserTab, ItemTab)
    return _tc_math(ur, ir, jr, fi, fj, E, Bp, Ic)


# speedup vs baseline: 1.3897x; 1.3897x over previous
import functools

import jax
import jax.numpy as jnp
from jax import lax
from jax.experimental import pallas as pl
from jax.experimental.pallas import tpu as pltpu
from jax.experimental.pallas import tpu_sc as plsc

N_ITEMS = 100000
N_CAT = 1000
FDIM = 512
K = 64
K2 = 32
B = 16384
LAMBDA_W = 0.01
LAMBDA_B = 0.01

NC, NS = 2, 16
NW = NC * NS
W = B // NW
CF = 64
CN = 128


def _sc_gather_f(i, j, F):
    mesh = plsc.VectorSubcoreMesh(core_axis_name="c", subcore_axis_name="s")
    f32 = jnp.float32
    out_type = (
        jax.ShapeDtypeStruct((B, FDIM), f32),
        jax.ShapeDtypeStruct((B, FDIM), f32),
    )
    scratch = [
        pltpu.VMEM((W,), jnp.int32),
        pltpu.VMEM((W,), jnp.int32),
        pltpu.VMEM((CF, FDIM), f32),
        pltpu.VMEM((CF, FDIM), f32),
        pltpu.SemaphoreType.DMA,
    ]

    @functools.partial(pl.kernel, out_type=out_type, mesh=mesh,
                       scratch_types=scratch)
    def body(i_h, j_h, F_h, fi_o, fj_o, iv, jv, fi_v, fj_v, sem):
        wid = lax.axis_index("s") * NC + lax.axis_index("c")
        base0 = wid * W
        d = [pltpu.async_copy(i_h.at[pl.ds(base0, W)], iv, sem),
             pltpu.async_copy(j_h.at[pl.ds(base0, W)], jv, sem)]
        for t in d:
            t.wait()

        def chunk(c, carry):
            o = c * CF
            s = pl.ds(o, CF)
            d = [pltpu.async_copy(F_h.at[iv.at[s]], fi_v, sem),
                 pltpu.async_copy(F_h.at[jv.at[s]], fj_v, sem)]
            for t in d:
                t.wait()
            so = pl.ds(base0 + o, CF)
            d = [pltpu.async_copy(fi_v, fi_o.at[so], sem),
                 pltpu.async_copy(fj_v, fj_o.at[so], sem)]
            for t in d:
                t.wait()
            return carry

        lax.fori_loop(0, W // CF, chunk, 0)

    return body(i, j, F)


def _sc_gather_narrow(u, i, j, Bi, Gu, Gi, Tu, Ic, IC):
    mesh = plsc.VectorSubcoreMesh(core_axis_name="c", subcore_axis_name="s")
    f32 = jnp.float32
    out_type = (
        jax.ShapeDtypeStruct((B, K), f32),
        jax.ShapeDtypeStruct((B, K), f32),
        jax.ShapeDtypeStruct((B, K), f32),
        jax.ShapeDtypeStruct((B, K2), f32),
        jax.ShapeDtypeStruct((B, K2), f32),
        jax.ShapeDtypeStruct((B, K2), f32),
        jax.ShapeDtypeStruct((B,), f32),
        jax.ShapeDtypeStruct((B,), f32),
    )
    scratch = [
        pltpu.VMEM((W,), jnp.int32),
        pltpu.VMEM((W,), jnp.int32),
        pltpu.VMEM((W,), jnp.int32),
        pltpu.VMEM((W,), jnp.int32),
        pltpu.VMEM((W,), jnp.int32),
        pltpu.VMEM((CN, K), f32),
        pltpu.VMEM((CN, K), f32),
        pltpu.VMEM((CN, K), f32),
        pltpu.VMEM((CN, K2), f32),
        pltpu.VMEM((CN, K2), f32),
        pltpu.VMEM((CN, K2), f32),
        pltpu.VMEM((CN,), f32),
        pltpu.VMEM((CN,), f32),
        pltpu.SemaphoreType.DMA,
    ]

    @functools.partial(pl.kernel, out_type=out_type, mesh=mesh,
                       scratch_types=scratch,
                       compiler_params=pltpu.CompilerParams(
                           use_tc_tiling_on_sc=False))
    def body(u_h, i_h, j_h, Bi_h, Gu_h, Gi_h, Tu_h, Ic_h, IC_h,
             gu_o, gi_o, gj_o, tu_o, cfi_o, cfj_o, bi_o, bj_o,
             uv, iv, jv, civ, cjv, gu_v, gi_v, gj_v,
             tu_v, cfi_v, cfj_v, bi_v, bj_v, sem):
        wid = lax.axis_index("s") * NC + lax.axis_index("c")
        base0 = wid * W
        d = [pltpu.async_copy(u_h.at[pl.ds(base0, W)], uv, sem),
             pltpu.async_copy(i_h.at[pl.ds(base0, W)], iv, sem),
             pltpu.async_copy(j_h.at[pl.ds(base0, W)], jv, sem)]
        for t in d:
            t.wait()
        d = []
        for k in range(W // CN):
            s = pl.ds(k * CN, CN)
            d.append(pltpu.async_copy(IC_h.at[iv.at[s]], civ.at[s], sem))
            d.append(pltpu.async_copy(IC_h.at[jv.at[s]], cjv.at[s], sem))
        for t in d:
            t.wait()

        def chunk(c, carry):
            o = c * CN
            s = pl.ds(o, CN)
            d = [
                pltpu.async_copy(Gu_h.at[uv.at[s]], gu_v, sem),
                pltpu.async_copy(Gi_h.at[iv.at[s]], gi_v, sem),
                pltpu.async_copy(Gi_h.at[jv.at[s]], gj_v, sem),
                pltpu.async_copy(Tu_h.at[uv.at[s]], tu_v, sem),
                pltpu.async_copy(Ic_h.at[civ.at[s]], cfi_v, sem),
                pltpu.async_copy(Ic_h.at[cjv.at[s]], cfj_v, sem),
                pltpu.async_copy(Bi_h.at[iv.at[s]], bi_v, sem),
                pltpu.async_copy(Bi_h.at[jv.at[s]], bj_v, sem),
            ]
            for t in d:
                t.wait()
            so = pl.ds(base0 + o, CN)
            d = [
                pltpu.async_copy(gu_v, gu_o.at[so], sem),
                pltpu.async_copy(gi_v, gi_o.at[so], sem),
                pltpu.async_copy(gj_v, gj_o.at[so], sem),
                pltpu.async_copy(tu_v, tu_o.at[so], sem),
                pltpu.async_copy(cfi_v, cfi_o.at[so], sem),
                pltpu.async_copy(cfj_v, cfj_o.at[so], sem),
                pltpu.async_copy(bi_v, bi_o.at[so], sem),
                pltpu.async_copy(bj_v, bj_o.at[so], sem),
            ]
            for t in d:
                t.wait()
            return carry

        lax.fori_loop(0, W // CN, chunk, 0)

    return body(u, i, j, Bi, Gu, Gi, Tu, Ic, IC)


def _tc_body(gu_r, gi_r, gj_r, tu_r, cfi_r, cfj_r, bi_r, bj_r,
             fi_r, fj_r, E_r, Bp_r, loss_r, auc_r):
    pid = pl.program_id(0)
    f32 = jnp.float32
    gu = gu_r[...]
    gi = gi_r[...]
    gj = gj_r[...]
    tu = tu_r[...]
    cfd = cfi_r[...] - cfj_r[...]
    bi = bi_r[...][:, None]
    bj = bj_r[...][:, None]

    fd = fi_r[...] - fj_r[...]
    t = jnp.dot(fd, E_r[...], preferred_element_type=f32)
    g = jnp.dot(fd, Bp_r[...], preferred_element_type=f32)

    ones_k = jnp.ones((K, 1), f32)
    ones_k2 = jnp.ones((K2, 1), f32)
    x = (bi - bj
         + jnp.dot(gu * (gi - gj), ones_k, preferred_element_type=f32)
         + jnp.dot(tu * t - tu * cfd, ones_k2, preferred_element_type=f32)
         + g)
    pll = jnp.minimum(x, 0.0) - jnp.log1p(jnp.exp(-jnp.abs(x)))
    pauc = (x > 0.0).astype(f32)
    preg = (0.5 * LAMBDA_W * (
                jnp.dot(gu * gu + gi * gi + gj * gj, ones_k,
                        preferred_element_type=f32)
                + jnp.dot(tu * tu, ones_k2, preferred_element_type=f32))
            + 0.5 * LAMBDA_B * (bi * bi + bj * bj))

    @pl.when(pid == 0)
    def _():
        loss_r[0, 0] = 0.0
        auc_r[0, 0] = 0.0

    loss_r[0, 0] += jnp.sum(preg - pll)
    auc_r[0, 0] += jnp.sum(pauc)


def _tc_math(gu, gi, gj, tu, cfi, cfj, bi, bj, fi, fj, E, Bp):
    BLK = 512
    G = B // BLK
    f32 = jnp.float32
    row = lambda b: (b, 0)
    full = lambda b: (0, 0)
    vec = lambda b: (b,)
    grid_spec = pl.GridSpec(
        grid=(G,),
        in_specs=[
            pl.BlockSpec((BLK, K), row),
            pl.BlockSpec((BLK, K), row),
            pl.BlockSpec((BLK, K), row),
            pl.BlockSpec((BLK, K2), row),
            pl.BlockSpec((BLK, K2), row),
            pl.BlockSpec((BLK, K2), row),
            pl.BlockSpec((BLK,), vec),
            pl.BlockSpec((BLK,), vec),
            pl.BlockSpec((BLK, FDIM), row),
            pl.BlockSpec((BLK, FDIM), row),
            pl.BlockSpec((FDIM, K2), full),
            pl.BlockSpec((FDIM, 1), full),
        ],
        out_specs=[
            pl.BlockSpec((1, 1), full, memory_space=pltpu.SMEM),
            pl.BlockSpec((1, 1), full, memory_space=pltpu.SMEM),
        ],
    )
    loss, auc = pl.pallas_call(
        _tc_body,
        grid_spec=grid_spec,
        out_shape=[jax.ShapeDtypeStruct((1, 1), f32),
                   jax.ShapeDtypeStruct((1, 1), f32)],
        compiler_params=pltpu.CompilerParams(
            dimension_semantics=("arbitrary",)),
    )(gu, gi, gj, tu, cfi, cfj, bi, bj, fi, fj, E, Bp)
    return loss[0, 0], auc[0, 0]


def kernel(u, i, j, Bi, Gu, Gi, Tu, Ic, E, Bp, F, IC):
    u = u.astype(jnp.int32)
    i = i.astype(jnp.int32)
    j = j.astype(jnp.int32)
    fi, fj = _sc_gather_f(i, j, F)
    gu, gi, gj, tu, cfi, cfj, bi, bj = _sc_gather_narrow(
        u, i, j, Bi, Gu, Gi, Tu, Ic, IC)
    return _tc_math(gu, gi, gj, tu, cfi, cfj, bi, bj, fi, fj, E, Bp)


# speedup vs baseline: 1.8513x; 1.3322x over previous
import functools

import jax
import jax.numpy as jnp
from jax import lax
from jax.experimental import pallas as pl
from jax.experimental.pallas import tpu as pltpu
from jax.experimental.pallas import tpu_sc as plsc

N_ITEMS = 100000
N_CAT = 1000
FDIM = 512
K = 64
K2 = 32
B = 16384
LAMBDA_W = 0.01
LAMBDA_B = 0.01

NC, NS = 2, 16
NW = NC * NS
W = B // NW
CF = 64
CT = 128
TAB = 128
PBLK = 2048
NPAD = 49 * PBLK


def _pack_body(gut_r, tut_r, git_r, bi_r, ic_r, ut_r, it_r):
    f32 = jnp.float32
    gu = jnp.transpose(gut_r[...], (1, 0))
    tu = jnp.transpose(tut_r[...], (1, 0))
    gi = jnp.transpose(git_r[...], (1, 0))
    blk = gu.shape[0]
    z32 = jnp.zeros((blk, TAB - K - K2), f32)
    ut_r[...] = jnp.concatenate([gu, tu, z32], axis=1)
    bi2 = bi_r[...][:, None]
    icf = jax.lax.bitcast_convert_type(ic_r[...], f32)[:, None]
    z62 = jnp.zeros((blk, TAB - K - 2), f32)
    it_r[...] = jnp.concatenate([gi, bi2, icf, z62], axis=1)


def _pack_tables(GuT, TuT, GiT, Bi, IC):
    f32 = jnp.float32
    G = NPAD // PBLK
    col = lambda b: (0, b)
    vec = lambda b: (b,)
    row = lambda b: (b, 0)
    return pl.pallas_call(
        _pack_body,
        grid=(G,),
        in_specs=[
            pl.BlockSpec((K, PBLK), col),
            pl.BlockSpec((K2, PBLK), col),
            pl.BlockSpec((K, PBLK), col),
            pl.BlockSpec((PBLK,), vec),
            pl.BlockSpec((PBLK,), vec),
        ],
        out_specs=[pl.BlockSpec((PBLK, TAB), row),
                   pl.BlockSpec((PBLK, TAB), row)],
        out_shape=[jax.ShapeDtypeStruct((NPAD, TAB), f32),
                   jax.ShapeDtypeStruct((NPAD, TAB), f32)],
        compiler_params=pltpu.CompilerParams(
            dimension_semantics=("arbitrary",)),
    )(GuT, TuT, GiT, Bi, IC)


def _sc_gather_f(i, j, F):
    mesh = plsc.VectorSubcoreMesh(core_axis_name="c", subcore_axis_name="s")
    f32 = jnp.float32
    out_type = (
        jax.ShapeDtypeStruct((B, FDIM), f32),
        jax.ShapeDtypeStruct((B, FDIM), f32),
    )
    scratch = [
        pltpu.VMEM((W,), jnp.int32),
        pltpu.VMEM((W,), jnp.int32),
        pltpu.VMEM((CF, FDIM), f32),
        pltpu.VMEM((CF, FDIM), f32),
        pltpu.SemaphoreType.DMA,
    ]

    @functools.partial(pl.kernel, out_type=out_type, mesh=mesh,
                       scratch_types=scratch)
    def body(i_h, j_h, F_h, fi_o, fj_o, iv, jv, fi_v, fj_v, sem):
        wid = lax.axis_index("s") * NC + lax.axis_index("c")
        base0 = wid * W
        d = [pltpu.async_copy(i_h.at[pl.ds(base0, W)], iv, sem),
             pltpu.async_copy(j_h.at[pl.ds(base0, W)], jv, sem)]
        for t in d:
            t.wait()

        def chunk(c, carry):
            o = c * CF
            s = pl.ds(o, CF)
            d = [pltpu.async_copy(F_h.at[iv.at[s]], fi_v, sem),
                 pltpu.async_copy(F_h.at[jv.at[s]], fj_v, sem)]
            for t in d:
                t.wait()
            so = pl.ds(base0 + o, CF)
            d = [pltpu.async_copy(fi_v, fi_o.at[so], sem),
                 pltpu.async_copy(fj_v, fj_o.at[so], sem)]
            for t in d:
                t.wait()
            return carry

        lax.fori_loop(0, W // CF, chunk, 0)

    return body(i, j, F)


def _sc_gather_tabs(u, i, j, UserTab, ItemTab):
    mesh = plsc.VectorSubcoreMesh(core_axis_name="c", subcore_axis_name="s")
    f32 = jnp.float32
    out_type = (
        jax.ShapeDtypeStruct((B, TAB), f32),
        jax.ShapeDtypeStruct((B, TAB), f32),
        jax.ShapeDtypeStruct((B, TAB), f32),
    )
    scratch = [
        pltpu.VMEM((W,), jnp.int32),
        pltpu.VMEM((W,), jnp.int32),
        pltpu.VMEM((W,), jnp.int32),
        pltpu.VMEM((CT, TAB), f32),
        pltpu.VMEM((CT, TAB), f32),
        pltpu.VMEM((CT, TAB), f32),
        pltpu.SemaphoreType.DMA,
    ]

    @functools.partial(pl.kernel, out_type=out_type, mesh=mesh,
                       scratch_types=scratch)
    def body(u_h, i_h, j_h, UT_h, IT_h,
             ur_o, ir_o, jr_o, uv, iv, jv, ur_v, ir_v, jr_v, sem):
        wid = lax.axis_index("s") * NC + lax.axis_index("c")
        base0 = wid * W
        d = [pltpu.async_copy(u_h.at[pl.ds(base0, W)], uv, sem),
             pltpu.async_copy(i_h.at[pl.ds(base0, W)], iv, sem),
             pltpu.async_copy(j_h.at[pl.ds(base0, W)], jv, sem)]
        for t in d:
            t.wait()

        def chunk(c, carry):
            o = c * CT
            s = pl.ds(o, CT)
            d = [pltpu.async_copy(UT_h.at[uv.at[s]], ur_v, sem),
                 pltpu.async_copy(IT_h.at[iv.at[s]], ir_v, sem),
                 pltpu.async_copy(IT_h.at[jv.at[s]], jr_v, sem)]
            for t in d:
                t.wait()
            so = pl.ds(base0 + o, CT)
            d = [pltpu.async_copy(ur_v, ur_o.at[so], sem),
                 pltpu.async_copy(ir_v, ir_o.at[so], sem),
                 pltpu.async_copy(jr_v, jr_o.at[so], sem)]
            for t in d:
                t.wait()
            return carry

        lax.fori_loop(0, W // CT, chunk, 0)

    return body(u, i, j, UserTab, ItemTab)


def _tc_body(ur_r, ir_r, jr_r, fi_r, fj_r, E_r, Bp_r, Ic_r, loss_r, auc_r):
    pid = pl.program_id(0)
    f32 = jnp.float32
    ub = ur_r[...]
    ib = ir_r[...]
    jb = jr_r[...]
    gu = ub[:, :K]
    tu = ub[:, K:K + K2]
    gi = ib[:, :K]
    gj = jb[:, :K]
    bi = ib[:, K:K + 1]
    bj = jb[:, K:K + 1]
    ci = jax.lax.bitcast_convert_type(ib[:, K + 1:K + 2], jnp.int32)
    cj = jax.lax.bitcast_convert_type(jb[:, K + 1:K + 2], jnp.int32)

    blk = ub.shape[0]
    cats = jax.lax.broadcasted_iota(jnp.int32, (blk, N_CAT), 1)
    zdiff = (cats == ci).astype(f32) - (cats == cj).astype(f32)
    cfd = jnp.dot(zdiff, Ic_r[...], preferred_element_type=f32)

    fd = fi_r[...] - fj_r[...]
    t = jnp.dot(fd, E_r[...], preferred_element_type=f32)
    g = jnp.dot(fd, Bp_r[...], preferred_element_type=f32)

    ones_k = jnp.ones((K, 1), f32)
    ones_k2 = jnp.ones((K2, 1), f32)
    x = (bi - bj
         + jnp.dot(gu * (gi - gj), ones_k, preferred_element_type=f32)
         + jnp.dot(tu * (t - cfd), ones_k2, preferred_element_type=f32)
         + g)
    pll = jnp.minimum(x, 0.0) - jnp.log1p(jnp.exp(-jnp.abs(x)))
    pauc = (x > 0.0).astype(f32)
    preg = (0.5 * LAMBDA_W * (
                jnp.dot(gu * gu + gi * gi + gj * gj, ones_k,
                        preferred_element_type=f32)
                + jnp.dot(tu * tu, ones_k2, preferred_element_type=f32))
            + 0.5 * LAMBDA_B * (bi * bi + bj * bj))

    @pl.when(pid == 0)
    def _():
        loss_r[0, 0] = 0.0
        auc_r[0, 0] = 0.0

    loss_r[0, 0] += jnp.sum(preg - pll)
    auc_r[0, 0] += jnp.sum(pauc)


def _tc_math(ur, ir, jr, fi, fj, E, Bp, Ic):
    BLK = 512
    G = B // BLK
    f32 = jnp.float32
    row = lambda b: (b, 0)
    full = lambda b: (0, 0)
    grid_spec = pl.GridSpec(
        grid=(G,),
        in_specs=[
            pl.BlockSpec((BLK, TAB), row),
            pl.BlockSpec((BLK, TAB), row),
            pl.BlockSpec((BLK, TAB), row),
            pl.BlockSpec((BLK, FDIM), row),
            pl.BlockSpec((BLK, FDIM), row),
            pl.BlockSpec((FDIM, K2), full),
            pl.BlockSpec((FDIM, 1), full),
            pl.BlockSpec((N_CAT, K2), full),
        ],
        out_specs=[
            pl.BlockSpec((1, 1), full, memory_space=pltpu.SMEM),
            pl.BlockSpec((1, 1), full, memory_space=pltpu.SMEM),
        ],
    )
    loss, auc = pl.pallas_call(
        _tc_body,
        grid_spec=grid_spec,
        out_shape=[jax.ShapeDtypeStruct((1, 1), f32),
                   jax.ShapeDtypeStruct((1, 1), f32)],
        compiler_params=pltpu.CompilerParams(
            dimension_semantics=("arbitrary",)),
    )(ur, ir, jr, fi, fj, E, Bp, Ic)
    return loss[0, 0], auc[0, 0]


def kernel(u, i, j, Bi, Gu, Gi, Tu, Ic, E, Bp, F, IC):
    u = u.astype(jnp.int32)
    i = i.astype(jnp.int32)
    j = j.astype(jnp.int32)
    fi, fj = _sc_gather_f(i, j, F)
    UserTab, ItemTab = _pack_tables(Gu.T, Tu.T, Gi.T, Bi, IC)
    ur, ir, jr = _sc_gather_tabs(u, i, j, UserTab, ItemTab)
    return _tc_math(ur, ir, jr, fi, fj, E, Bp, Ic)


# speedup vs baseline: 2.2497x; 1.2152x over previous
import functools

import jax
import jax.numpy as jnp
from jax import lax
from jax.experimental import pallas as pl
from jax.experimental.pallas import tpu as pltpu
from jax.experimental.pallas import tpu_sc as plsc

N_ITEMS = 100000
N_CAT = 1000
FDIM = 512
K = 64
K2 = 32
B = 16384
LAMBDA_W = 0.01
LAMBDA_B = 0.01

NC, NS = 2, 16
NW = NC * NS
W = B // NW
CF = 64
CT = 128
TAB = 128
PBLK = 2048
NPAD = 49 * PBLK


def _pack_body(gut_r, tut_r, git_r, bi_r, ic_r, ut_r, it_r):
    f32 = jnp.float32
    gu = jnp.transpose(gut_r[...], (1, 0))
    tu = jnp.transpose(tut_r[...], (1, 0))
    gi = jnp.transpose(git_r[...], (1, 0))
    blk = gu.shape[0]
    z32 = jnp.zeros((blk, TAB - K - K2), f32)
    ut_r[...] = jnp.concatenate([gu, tu, z32], axis=1)
    bi2 = bi_r[...][:, None]
    icf = jax.lax.bitcast_convert_type(ic_r[...], f32)[:, None]
    z62 = jnp.zeros((blk, TAB - K - 2), f32)
    it_r[...] = jnp.concatenate([gi, bi2, icf, z62], axis=1)


def _pack_tables(GuT, TuT, GiT, Bi, IC):
    f32 = jnp.float32
    G = NPAD // PBLK
    col = lambda b: (0, b)
    vec = lambda b: (b,)
    row = lambda b: (b, 0)
    return pl.pallas_call(
        _pack_body,
        grid=(G,),
        in_specs=[
            pl.BlockSpec((K, PBLK), col),
            pl.BlockSpec((K2, PBLK), col),
            pl.BlockSpec((K, PBLK), col),
            pl.BlockSpec((PBLK,), vec),
            pl.BlockSpec((PBLK,), vec),
        ],
        out_specs=[pl.BlockSpec((PBLK, TAB), row),
                   pl.BlockSpec((PBLK, TAB), row)],
        out_shape=[jax.ShapeDtypeStruct((NPAD, TAB), f32),
                   jax.ShapeDtypeStruct((NPAD, TAB), f32)],
        compiler_params=pltpu.CompilerParams(
            dimension_semantics=("arbitrary",)),
    )(GuT, TuT, GiT, Bi, IC)


def _sc_gather_f(i, j, F):
    mesh = plsc.VectorSubcoreMesh(core_axis_name="c", subcore_axis_name="s")
    f32 = jnp.float32
    out_type = jax.ShapeDtypeStruct((B, FDIM), f32)
    scratch = [
        pltpu.VMEM((W,), jnp.int32),
        pltpu.VMEM((W,), jnp.int32),
        pltpu.VMEM((CF, FDIM), f32),
        pltpu.VMEM((CF, FDIM), f32),
        pltpu.SemaphoreType.DMA,
    ]

    @functools.partial(pl.kernel, out_type=out_type, mesh=mesh,
                       scratch_types=scratch)
    def body(i_h, j_h, F_h, fd_o, iv, jv, fi_v, fj_v, sem):
        wid = lax.axis_index("s") * NC + lax.axis_index("c")
        base0 = wid * W
        d = [pltpu.async_copy(i_h.at[pl.ds(base0, W)], iv, sem),
             pltpu.async_copy(j_h.at[pl.ds(base0, W)], jv, sem)]
        for t in d:
            t.wait()

        def chunk(c, carry):
            o = c * CF
            s = pl.ds(o, CF)
            d = [pltpu.async_copy(F_h.at[iv.at[s]], fi_v, sem),
                 pltpu.async_copy(F_h.at[jv.at[s]], fj_v, sem)]
            for t in d:
                t.wait()

            def row(r, cc):
                for k in range(FDIM // 16):
                    sl = pl.ds(k * 16, 16)
                    fi_v[r, sl] = fi_v[r, sl] - fj_v[r, sl]
                return cc

            lax.fori_loop(0, CF, row, 0)
            pltpu.async_copy(fi_v, fd_o.at[pl.ds(base0 + o, CF)], sem).wait()
            return carry

        lax.fori_loop(0, W // CF, chunk, 0)

    return body(i, j, F)


def _sc_gather_tabs(u, i, j, UserTab, ItemTab, fd):
    mesh = plsc.VectorSubcoreMesh(core_axis_name="c", subcore_axis_name="s")
    f32 = jnp.float32
    out_type = (
        jax.ShapeDtypeStruct((B, TAB), f32),
        jax.ShapeDtypeStruct((B, TAB), f32),
        jax.ShapeDtypeStruct((B, TAB), f32),
    )
    scratch = [
        pltpu.VMEM((W,), jnp.int32),
        pltpu.VMEM((W,), jnp.int32),
        pltpu.VMEM((W,), jnp.int32),
        pltpu.VMEM((CT, TAB), f32),
        pltpu.VMEM((CT, TAB), f32),
        pltpu.VMEM((CT, TAB), f32),
        pltpu.SemaphoreType.DMA,
    ]

    @functools.partial(pl.kernel, out_type=out_type, mesh=mesh,
                       scratch_types=scratch)
    def body(u_h, i_h, j_h, UT_h, IT_h, fd_h,
             ur_o, ir_o, jr_o, uv, iv, jv, ur_v, ir_v, jr_v, sem):
        del fd_h
        wid = lax.axis_index("s") * NC + lax.axis_index("c")
        base0 = wid * W
        d = [pltpu.async_copy(u_h.at[pl.ds(base0, W)], uv, sem),
             pltpu.async_copy(i_h.at[pl.ds(base0, W)], iv, sem),
             pltpu.async_copy(j_h.at[pl.ds(base0, W)], jv, sem)]
        for t in d:
            t.wait()

        def chunk(c, carry):
            o = c * CT
            s = pl.ds(o, CT)
            d = [pltpu.async_copy(UT_h.at[uv.at[s]], ur_v, sem),
                 pltpu.async_copy(IT_h.at[iv.at[s]], ir_v, sem),
                 pltpu.async_copy(IT_h.at[jv.at[s]], jr_v, sem)]
            for t in d:
                t.wait()
            so = pl.ds(base0 + o, CT)
            d = [pltpu.async_copy(ur_v, ur_o.at[so], sem),
                 pltpu.async_copy(ir_v, ir_o.at[so], sem),
                 pltpu.async_copy(jr_v, jr_o.at[so], sem)]
            for t in d:
                t.wait()
            return carry

        lax.fori_loop(0, W // CT, chunk, 0)

    return body(u, i, j, UserTab, ItemTab, fd)


def _tc_body(ur_r, ir_r, jr_r, fd_r, E_r, Bp_r, Ic_r, loss_r, auc_r):
    pid = pl.program_id(0)
    f32 = jnp.float32
    ub = ur_r[...]
    ib = ir_r[...]
    jb = jr_r[...]
    gu = ub[:, :K]
    tu = ub[:, K:K + K2]
    gi = ib[:, :K]
    gj = jb[:, :K]
    bi = ib[:, K:K + 1]
    bj = jb[:, K:K + 1]
    ci = jax.lax.bitcast_convert_type(ib[:, K + 1:K + 2], jnp.int32)
    cj = jax.lax.bitcast_convert_type(jb[:, K + 1:K + 2], jnp.int32)

    blk = ub.shape[0]
    cats = jax.lax.broadcasted_iota(jnp.int32, (blk, N_CAT), 1)
    zdiff = (cats == ci).astype(f32) - (cats == cj).astype(f32)
    cfd = jnp.dot(zdiff, Ic_r[...], preferred_element_type=f32)

    fd = fd_r[...]
    t = jnp.dot(fd, E_r[...], preferred_element_type=f32)
    g = jnp.dot(fd, Bp_r[...], preferred_element_type=f32)

    ones_k = jnp.ones((K, 1), f32)
    ones_k2 = jnp.ones((K2, 1), f32)
    x = (bi - bj
         + jnp.dot(gu * (gi - gj), ones_k, preferred_element_type=f32)
         + jnp.dot(tu * (t - cfd), ones_k2, preferred_element_type=f32)
         + g)
    pll = jnp.minimum(x, 0.0) - jnp.log1p(jnp.exp(-jnp.abs(x)))
    pauc = (x > 0.0).astype(f32)
    preg = (0.5 * LAMBDA_W * (
                jnp.dot(gu * gu + gi * gi + gj * gj, ones_k,
                        preferred_element_type=f32)
                + jnp.dot(tu * tu, ones_k2, preferred_element_type=f32))
            + 0.5 * LAMBDA_B * (bi * bi + bj * bj))

    @pl.when(pid == 0)
    def _():
        loss_r[0, 0] = 0.0
        auc_r[0, 0] = 0.0

    loss_r[0, 0] += jnp.sum(preg - pll)
    auc_r[0, 0] += jnp.sum(pauc)


def _tc_math(ur, ir, jr, fd, E, Bp, Ic):
    BLK = 512
    G = B // BLK
    f32 = jnp.float32
    row = lambda b: (b, 0)
    full = lambda b: (0, 0)
    grid_spec = pl.GridSpec(
        grid=(G,),
        in_specs=[
            pl.BlockSpec((BLK, TAB), row),
            pl.BlockSpec((BLK, TAB), row),
            pl.BlockSpec((BLK, TAB), row),
            pl.BlockSpec((BLK, FDIM), row),
            pl.BlockSpec((FDIM, K2), full),
            pl.BlockSpec((FDIM, 1), full),
            pl.BlockSpec((N_CAT, K2), full),
        ],
        out_specs=[
            pl.BlockSpec((1, 1), full, memory_space=pltpu.SMEM),
            pl.BlockSpec((1, 1), full, memory_space=pltpu.SMEM),
        ],
    )
    loss, auc = pl.pallas_call(
        _tc_body,
        grid_spec=grid_spec,
        out_shape=[jax.ShapeDtypeStruct((1, 1), f32),
                   jax.ShapeDtypeStruct((1, 1), f32)],
        compiler_params=pltpu.CompilerParams(
            dimension_semantics=("arbitrary",)),
    )(ur, ir, jr, fd, E, Bp, Ic)
    return loss[0, 0], auc[0, 0]


def kernel(u, i, j, Bi, Gu, Gi, Tu, Ic, E, Bp, F, IC):
    u = u.astype(jnp.int32)
    i = i.astype(jnp.int32)
    j = j.astype(jnp.int32)
    fd = _sc_gather_f(i, j, F)
    UserTab, ItemTab = _pack_tables(Gu.T, Tu.T, Gi.T, Bi, IC)
    ur, ir, jr = _sc_gather_tabs(u, i, j, UserTab, ItemTab, fd)
    return _tc_math(ur, ir, jr, fd, E, Bp, Ic)
